# baseline (speedup 1.0000x reference)
import functools

import jax
import jax.numpy as jnp
from jax import lax
from jax.experimental import pallas as pl
from jax.experimental.pallas import tpu as pltpu
from jax.experimental.pallas import tpu_sc as plsc

ENT_NUM = 100000
EMB_DIM = 64
BATCH = 128
PI = 3.141592653589793

E_BLK = 8192
E_CHK = 512
N_EBLK = (ENT_NUM + E_BLK - 1) // E_BLK
B_GRP = 8
N_BGRP = BATCH // B_GRP

_NC, _NS = 2, 16
_ROWS_PER_W = BATCH // (_NC * _NS) * 2


def _sc_gather_body(ent_hbm, hidx_hbm, rel_hbm, ridx_hbm, h_out, r_out,
                    hidx_v, hrow_v, ridx_v, rrow_v, sem):
    wid = lax.axis_index("s") * _NC + lax.axis_index("c")

    @pl.when(wid < _NS)
    def _():
        base = wid * _ROWS_PER_W
        pltpu.sync_copy(hidx_hbm.at[pl.ds(base, _ROWS_PER_W)], hidx_v)
        pltpu.async_copy(ent_hbm.at[hidx_v], hrow_v, sem).wait()
        pltpu.sync_copy(hrow_v, h_out.at[pl.ds(base, _ROWS_PER_W)])

    @pl.when(wid >= _NS)
    def _():
        base = (wid - _NS) * _ROWS_PER_W
        pltpu.sync_copy(ridx_hbm.at[pl.ds(base, _ROWS_PER_W)], ridx_v)
        pltpu.async_copy(rel_hbm.at[ridx_v], rrow_v, sem).wait()
        pltpu.sync_copy(rrow_v, r_out.at[pl.ds(base, _ROWS_PER_W)])


@functools.cache
def _get_sc_gather():
    return functools.partial(
        pl.kernel,
        out_type=(
            jax.ShapeDtypeStruct((BATCH, 2 * EMB_DIM), jnp.float32),
            jax.ShapeDtypeStruct((BATCH, 2 * EMB_DIM), jnp.float32),
        ),
        mesh=plsc.VectorSubcoreMesh(core_axis_name="c", subcore_axis_name="s",
                                    num_cores=_NC, num_subcores=_NS),
        scratch_types=[
            pltpu.VMEM((_ROWS_PER_W,), jnp.int32),
            pltpu.VMEM((_ROWS_PER_W, 2 * EMB_DIM), jnp.float32),
            pltpu.VMEM((_ROWS_PER_W,), jnp.int32),
            pltpu.VMEM((_ROWS_PER_W, 2 * EMB_DIM), jnp.float32),
            pltpu.SemaphoreType.DMA,
        ],
    )(_sc_gather_body)


def _pred_body(h_ref, r_ref, predt_ref):
    h = h_ref[...]
    re = h[:, :EMB_DIM]
    im = h[:, EMB_DIM:]
    den = jnp.sqrt(re * re + im * im) + 1e-08
    re = re / den
    im = im / den
    r = r_ref[...][:, :EMB_DIM] / (EMB_DIM / PI)
    cr = jnp.cos(r)
    sr = jnp.sin(r)
    re_hr = re * cr - im * sr
    im_hr = re * sr + im * cr
    predt_ref[...] = jnp.concatenate([re_hr, im_hr], axis=-1)


def _score_body(ent_ref, predt_ref, out_ref):
    et = ent_ref[...].T
    re = et[:EMB_DIM, :]
    im = et[EMB_DIM:, :]
    den = jnp.sqrt(re * re + im * im) + 1e-08
    tn = jnp.concatenate([re / den, im / den], axis=0)

    p = predt_ref[...].T
    ones = jnp.ones((1, 2 * EMB_DIM), jnp.float32)
    for c in range(E_BLK // E_CHK):
        tnc = tn[:, c * E_CHK:(c + 1) * E_CHK]
        rows = []
        for jj in range(BATCH):
            x = jnp.abs(tnc - p[:, jj:jj + 1])
            rows.append(jnp.dot(ones, x, preferred_element_type=jnp.float32))
        s = jnp.concatenate(rows, axis=0)
        out_ref[:, c * E_CHK:(c + 1) * E_CHK] = jax.nn.sigmoid(9.0 - s)


def kernel(head, ent_emb, rel_emb):
    h_idx = head[:, 0]
    r_idx = head[:, 1]
    rel_pad = jnp.pad(rel_emb, ((0, 0), (0, EMB_DIM)))
    h_rows, r_rows = _get_sc_gather()(ent_emb, h_idx, rel_pad, r_idx)

    predt = pl.pallas_call(
        _pred_body,
        out_shape=jax.ShapeDtypeStruct((BATCH, 2 * EMB_DIM), jnp.float32),
    )(h_rows, r_rows)

    out = pl.pallas_call(
        _score_body,
        grid=(N_EBLK,),
        in_specs=[
            pl.BlockSpec((E_BLK, 2 * EMB_DIM), lambda i: (i, 0)),
            pl.BlockSpec((BATCH, 2 * EMB_DIM), lambda i: (0, 0)),
        ],
        out_specs=pl.BlockSpec((BATCH, E_BLK), lambda i: (0, i)),
        out_shape=jax.ShapeDtypeStruct((BATCH, ENT_NUM), jnp.float32),
    )(ent_emb, predt)
    return out


# speedup vs baseline: 1.2660x; 1.2660x over previous
import functools

import jax
import jax.numpy as jnp
from jax import lax
from jax.experimental import pallas as pl
from jax.experimental.pallas import tpu as pltpu
from jax.experimental.pallas import tpu_sc as plsc

ENT_NUM = 100000
EMB_DIM = 64
BATCH = 128
PI = 3.141592653589793

E_BLK = 3584
E_CHK = 512
N_EBLK = (ENT_NUM + E_BLK - 1) // E_BLK
B_GRP = 8
N_BGRP = BATCH // B_GRP

_NC, _NS = 2, 16
_ROWS_PER_W = BATCH // (_NC * _NS) * 2


def _sc_gather_body(ent_hbm, hidx_hbm, rel_hbm, ridx_hbm, h_out, r_out,
                    hidx_v, hrow_v, ridx_v, rrow_v, sem):
    wid = lax.axis_index("s") * _NC + lax.axis_index("c")

    @pl.when(wid < _NS)
    def _():
        base = wid * _ROWS_PER_W
        pltpu.sync_copy(hidx_hbm.at[pl.ds(base, _ROWS_PER_W)], hidx_v)
        pltpu.async_copy(ent_hbm.at[hidx_v], hrow_v, sem).wait()
        pltpu.sync_copy(hrow_v, h_out.at[pl.ds(base, _ROWS_PER_W)])

    @pl.when(wid >= _NS)
    def _():
        base = (wid - _NS) * _ROWS_PER_W
        pltpu.sync_copy(ridx_hbm.at[pl.ds(base, _ROWS_PER_W)], ridx_v)
        pltpu.async_copy(rel_hbm.at[ridx_v], rrow_v, sem).wait()
        pltpu.sync_copy(rrow_v, r_out.at[pl.ds(base, _ROWS_PER_W)])


@functools.cache
def _get_sc_gather():
    return functools.partial(
        pl.kernel,
        out_type=(
            jax.ShapeDtypeStruct((BATCH, 2 * EMB_DIM), jnp.float32),
            jax.ShapeDtypeStruct((BATCH, 2 * EMB_DIM), jnp.float32),
        ),
        mesh=plsc.VectorSubcoreMesh(core_axis_name="c", subcore_axis_name="s",
                                    num_cores=_NC, num_subcores=_NS),
        scratch_types=[
            pltpu.VMEM((_ROWS_PER_W,), jnp.int32),
            pltpu.VMEM((_ROWS_PER_W, 2 * EMB_DIM), jnp.float32),
            pltpu.VMEM((_ROWS_PER_W,), jnp.int32),
            pltpu.VMEM((_ROWS_PER_W, 2 * EMB_DIM), jnp.float32),
            pltpu.SemaphoreType.DMA,
        ],
    )(_sc_gather_body)


def _pred_body(h_ref, r_ref, predt_ref):
    h = h_ref[...]
    re = h[:, :EMB_DIM]
    im = h[:, EMB_DIM:]
    den = jnp.sqrt(re * re + im * im) + 1e-08
    re = re / den
    im = im / den
    r = r_ref[...][:, :EMB_DIM] / (EMB_DIM / PI)
    cr = jnp.cos(r)
    sr = jnp.sin(r)
    re_hr = re * cr - im * sr
    im_hr = re * sr + im * cr
    predt_ref[...] = jnp.concatenate([re_hr, im_hr], axis=-1)


def _score_body(ent_ref, predt_ref, out_ref):
    et = ent_ref[...].T
    re = et[:EMB_DIM, :]
    im = et[EMB_DIM:, :]
    den = jnp.sqrt(re * re + im * im) + 1e-08
    tn = jnp.concatenate([re / den, im / den], axis=0)

    p = predt_ref[...].T
    ones = jnp.ones((1, 2 * EMB_DIM), jnp.float32)
    for c in range(E_BLK // E_CHK):
        tnc = tn[:, c * E_CHK:(c + 1) * E_CHK]
        rows = []
        for jj in range(BATCH):
            x = jnp.abs(tnc - p[:, jj:jj + 1])
            rows.append(jnp.dot(ones, x, preferred_element_type=jnp.float32))
        s = jnp.concatenate(rows, axis=0)
        out_ref[:, c * E_CHK:(c + 1) * E_CHK] = jax.nn.sigmoid(9.0 - s)


def kernel(head, ent_emb, rel_emb):
    h_idx = head[:, 0]
    r_idx = head[:, 1]
    rel_pad = jnp.pad(rel_emb, ((0, 0), (0, EMB_DIM)))
    h_rows, r_rows = _get_sc_gather()(ent_emb, h_idx, rel_pad, r_idx)

    predt = pl.pallas_call(
        _pred_body,
        out_shape=jax.ShapeDtypeStruct((BATCH, 2 * EMB_DIM), jnp.float32),
    )(h_rows, r_rows)

    out = pl.pallas_call(
        _score_body,
        grid=(N_EBLK,),
        in_specs=[
            pl.BlockSpec((E_BLK, 2 * EMB_DIM), lambda i: (i, 0)),
            pl.BlockSpec((BATCH, 2 * EMB_DIM), lambda i: (0, 0)),
        ],
        out_specs=pl.BlockSpec((BATCH, E_BLK), lambda i: (0, i)),
        out_shape=jax.ShapeDtypeStruct((BATCH, ENT_NUM), jnp.float32),
    )(ent_emb, predt)
    return out


# speedup vs baseline: 1.3060x; 1.0317x over previous
import functools

import jax
import jax.numpy as jnp
from jax import lax
from jax.experimental import pallas as pl
from jax.experimental.pallas import tpu as pltpu
from jax.experimental.pallas import tpu_sc as plsc

ENT_NUM = 100000
EMB_DIM = 64
BATCH = 128
PI = 3.141592653589793

E_BLK = 3584
E_CHK = 1792
N_EBLK = (ENT_NUM + E_BLK - 1) // E_BLK
B_GRP = 8
N_BGRP = BATCH // B_GRP

_NC, _NS = 2, 16
_ROWS_PER_W = BATCH // (_NC * _NS) * 2


def _sc_gather_body(ent_hbm, hidx_hbm, rel_hbm, ridx_hbm, h_out, r_out,
                    hidx_v, hrow_v, ridx_v, rrow_v, sem):
    wid = lax.axis_index("s") * _NC + lax.axis_index("c")

    @pl.when(wid < _NS)
    def _():
        base = wid * _ROWS_PER_W
        pltpu.sync_copy(hidx_hbm.at[pl.ds(base, _ROWS_PER_W)], hidx_v)
        pltpu.async_copy(ent_hbm.at[hidx_v], hrow_v, sem).wait()
        pltpu.sync_copy(hrow_v, h_out.at[pl.ds(base, _ROWS_PER_W)])

    @pl.when(wid >= _NS)
    def _():
        base = (wid - _NS) * _ROWS_PER_W
        pltpu.sync_copy(ridx_hbm.at[pl.ds(base, _ROWS_PER_W)], ridx_v)
        pltpu.async_copy(rel_hbm.at[ridx_v], rrow_v, sem).wait()
        pltpu.sync_copy(rrow_v, r_out.at[pl.ds(base, _ROWS_PER_W)])


@functools.cache
def _get_sc_gather():
    return functools.partial(
        pl.kernel,
        out_type=(
            jax.ShapeDtypeStruct((BATCH, 2 * EMB_DIM), jnp.float32),
            jax.ShapeDtypeStruct((BATCH, 2 * EMB_DIM), jnp.float32),
        ),
        mesh=plsc.VectorSubcoreMesh(core_axis_name="c", subcore_axis_name="s",
                                    num_cores=_NC, num_subcores=_NS),
        scratch_types=[
            pltpu.VMEM((_ROWS_PER_W,), jnp.int32),
            pltpu.VMEM((_ROWS_PER_W, 2 * EMB_DIM), jnp.float32),
            pltpu.VMEM((_ROWS_PER_W,), jnp.int32),
            pltpu.VMEM((_ROWS_PER_W, 2 * EMB_DIM), jnp.float32),
            pltpu.SemaphoreType.DMA,
        ],
    )(_sc_gather_body)


def _pred_body(h_ref, r_ref, predt_ref):
    h = h_ref[...]
    re = h[:, :EMB_DIM]
    im = h[:, EMB_DIM:]
    den = jnp.sqrt(re * re + im * im) + 1e-08
    re = re / den
    im = im / den
    r = r_ref[...][:, :EMB_DIM] / (EMB_DIM / PI)
    cr = jnp.cos(r)
    sr = jnp.sin(r)
    re_hr = re * cr - im * sr
    im_hr = re * sr + im * cr
    predt_ref[...] = jnp.concatenate([re_hr, im_hr], axis=-1)


def _score_body(ent_ref, predt_ref, out_ref):
    et = ent_ref[...].T
    re = et[:EMB_DIM, :]
    im = et[EMB_DIM:, :]
    den = jnp.sqrt(re * re + im * im) + 1e-08
    tn = jnp.concatenate([re / den, im / den], axis=0)

    p = predt_ref[...].T
    ones = jnp.ones((1, 2 * EMB_DIM), jnp.float32)
    for c in range(E_BLK // E_CHK):
        tnc = tn[:, c * E_CHK:(c + 1) * E_CHK]
        rows = []
        for jj in range(BATCH):
            x = jnp.abs(tnc - p[:, jj:jj + 1])
            rows.append(jnp.dot(ones, x, preferred_element_type=jnp.float32))
        s = jnp.concatenate(rows, axis=0)
        out_ref[:, c * E_CHK:(c + 1) * E_CHK] = jax.nn.sigmoid(9.0 - s)


def kernel(head, ent_emb, rel_emb):
    h_idx = head[:, 0]
    r_idx = head[:, 1]
    rel_pad = jnp.pad(rel_emb, ((0, 0), (0, EMB_DIM)))
    h_rows, r_rows = _get_sc_gather()(ent_emb, h_idx, rel_pad, r_idx)

    predt = pl.pallas_call(
        _pred_body,
        out_shape=jax.ShapeDtypeStruct((BATCH, 2 * EMB_DIM), jnp.float32),
    )(h_rows, r_rows)

    out = pl.pallas_call(
        _score_body,
        grid=(N_EBLK,),
        in_specs=[
            pl.BlockSpec((E_BLK, 2 * EMB_DIM), lambda i: (i, 0)),
            pl.BlockSpec((BATCH, 2 * EMB_DIM), lambda i: (0, 0)),
        ],
        out_specs=pl.BlockSpec((BATCH, E_BLK), lambda i: (0, i)),
        out_shape=jax.ShapeDtypeStruct((BATCH, ENT_NUM), jnp.float32),
    )(ent_emb, predt)
    return out
